# Initial kernel scaffold; baseline (speedup 1.0000x reference)
import jax
import jax.numpy as jnp
from jax.experimental import pallas as pl


def kernel(input1, input2, input3, input4, input5, input6, inputs7, inputs8, emb1, emb2, emb3, emb4, emb5, emb6, W_ih, W_hh, b_ih, b_hh, W_out, b_out):
    raise NotImplementedError("write your pallas kernel here")


# speedup vs baseline: 2.1409x; 2.1409x over previous
import functools

import jax
import jax.numpy as jnp
from jax import lax
from jax.experimental import pallas as pl
from jax.experimental.pallas import tpu as pltpu
from jax.experimental.pallas import tpu_sc as plsc

B, L, D, H, V = 256, 50, 128, 1024, 1000
N = B * L
NT = 6
TOT = NT * N
G3 = 3 * H

NW = 32
CH, CW = 20, 120


def _sc_gather(emb_all, idx3):
    mesh = plsc.VectorSubcoreMesh(core_axis_name="c", subcore_axis_name="s")
    info = plsc.get_sparse_core_info()
    nc = info.num_cores

    @functools.partial(
        pl.kernel,
        mesh=mesh,
        out_type=jax.ShapeDtypeStruct((TOT, D), jnp.float32),
        scratch_types=[
            pltpu.VMEM((CH, CW), jnp.int32),
            pltpu.VMEM((CW, D), jnp.float32),
            pltpu.VMEM((CW, D), jnp.float32),
            pltpu.SemaphoreType.DMA,
            pltpu.SemaphoreType.DMA,
        ],
    )
    def k(emb_hbm, idx_hbm, out_hbm, idx_v, buf0, buf1, sem0, sem1):
        wid = lax.axis_index("s") * nc + lax.axis_index("c")
        base = wid * (CH * CW)
        pltpu.sync_copy(idx_hbm.at[wid], idx_v)
        bufs = (buf0, buf1)
        sems = (sem0, sem1)
        cps = [None, None]
        cps[0] = pltpu.async_copy(emb_hbm.at[idx_v.at[0]], buf0, sem0)
        for j in range(CH):
            b = j & 1
            cps[b].wait()
            if j + 1 < CH:
                nb = (j + 1) & 1
                cps[nb] = pltpu.async_copy(
                    emb_hbm.at[idx_v.at[j + 1]], bufs[nb], sems[nb])
            pltpu.sync_copy(bufs[b], out_hbm.at[pl.ds(base + j * CW, CW)])

    return k(emb_all, idx3)


TB = 512


def _inproj(e_all, f78, WeT, WfT, b_ih2):
    def body(e_ref, f_ref, we_ref, wf_ref, b_ref, out_ref):
        acc = jnp.dot(e_ref[0], we_ref[0], preferred_element_type=jnp.float32)
        for i in range(1, NT):
            acc = acc + jnp.dot(e_ref[i], we_ref[i],
                                preferred_element_type=jnp.float32)
        f = f_ref[...]
        acc = acc + f[:, 0:1] * wf_ref[0:1, :]
        acc = acc + f[:, 1:2] * wf_ref[1:2, :]
        out_ref[...] = acc + b_ref[...]

    return pl.pallas_call(
        body,
        grid=(N // TB,),
        in_specs=[
            pl.BlockSpec((NT, TB, D), lambda j: (0, j, 0)),
            pl.BlockSpec((TB, 2), lambda j: (j, 0)),
            pl.BlockSpec((NT, D, G3), lambda j: (0, 0, 0)),
            pl.BlockSpec((2, G3), lambda j: (0, 0)),
            pl.BlockSpec((1, G3), lambda j: (0, 0)),
        ],
        out_specs=pl.BlockSpec((TB, G3), lambda j: (j, 0)),
        out_shape=jax.ShapeDtypeStruct((N, G3), jnp.float32),
    )(e_all, f78, WeT, WfT, b_ih2)


def _gru(gi3, WhhT, WoutT, b_hh2, b_out2):
    def body(gi_ref, whh_ref, bhh_ref, wout_ref, bo_ref, out_ref, h_ref):
        t = pl.program_id(0)

        @pl.when(t == 0)
        def _():
            h_ref[...] = jnp.zeros_like(h_ref)

        h = h_ref[...]
        gh = jnp.dot(h, whh_ref[...],
                     preferred_element_type=jnp.float32) + bhh_ref[...]
        g = gi_ref[0]
        r = jax.nn.sigmoid(g[:, :H] + gh[:, :H])
        z = jax.nn.sigmoid(g[:, H:2 * H] + gh[:, H:2 * H])
        n = jnp.tanh(g[:, 2 * H:] + r * gh[:, 2 * H:])
        h_new = (1.0 - z) * n + z * h
        h_ref[...] = h_new
        out_ref[0, :, :] = jnp.dot(
            h_new, wout_ref[...], preferred_element_type=jnp.float32
        ) + bo_ref[...]

    return pl.pallas_call(
        body,
        grid=(L,),
        in_specs=[
            pl.BlockSpec((1, B, G3), lambda t: (t, 0, 0)),
            pl.BlockSpec((H, G3), lambda t: (0, 0)),
            pl.BlockSpec((1, G3), lambda t: (0, 0)),
            pl.BlockSpec((H, V), lambda t: (0, 0)),
            pl.BlockSpec((1, V), lambda t: (0, 0)),
        ],
        out_specs=pl.BlockSpec((1, B, V), lambda t: (t, 0, 0)),
        out_shape=jax.ShapeDtypeStruct((L, B, V), jnp.float32),
        scratch_shapes=[pltpu.VMEM((B, H), jnp.float32)],
    )(gi3, WhhT, b_hh2, WoutT, b_out2)


def kernel(input1, input2, input3, input4, input5, input6, inputs7, inputs8,
           emb1, emb2, emb3, emb4, emb5, emb6,
           W_ih, W_hh, b_ih, b_hh, W_out, b_out):
    idx = jnp.stack([input1, input2, input3, input4, input5, input6])
    idx = idx.astype(jnp.int32).transpose(0, 2, 1).reshape(NT, N)
    idx = idx + jnp.arange(NT, dtype=jnp.int32)[:, None] * V
    idx3 = idx.reshape(NW, CH, CW)

    emb_all = jnp.concatenate([emb1, emb2, emb3, emb4, emb5, emb6], axis=0)
    e_all = _sc_gather(emb_all, idx3).reshape(NT, N, D)

    f78 = jnp.stack(
        [inputs7.T.reshape(N), inputs8.T.reshape(N)], axis=1)

    WeT = W_ih[:, :NT * D].T.reshape(NT, D, G3)
    WfT = W_ih[:, NT * D:].T
    gi = _inproj(e_all, f78, WeT, WfT, b_ih.reshape(1, G3))

    logits = _gru(gi.reshape(L, B, G3), W_hh.T, W_out.T,
                  b_hh.reshape(1, G3), b_out.reshape(1, V))
    return logits.transpose(1, 0, 2)


# speedup vs baseline: 2.1710x; 1.0140x over previous
import functools

import jax
import jax.numpy as jnp
from jax import lax
from jax.experimental import pallas as pl
from jax.experimental.pallas import tpu as pltpu
from jax.experimental.pallas import tpu_sc as plsc

B, L, D, H, V = 256, 50, 128, 1024, 1000
N = B * L
NT = 6
TOT = NT * N
G3 = 3 * H

NW = 32
CH, CW = 20, 120


def _sc_gather(emb_all, idx3):
    mesh = plsc.VectorSubcoreMesh(core_axis_name="c", subcore_axis_name="s")
    info = plsc.get_sparse_core_info()
    nc = info.num_cores

    @functools.partial(
        pl.kernel,
        mesh=mesh,
        out_type=jax.ShapeDtypeStruct((TOT, D), jnp.float32),
        scratch_types=[
            pltpu.VMEM((CH, CW), jnp.int32),
            pltpu.VMEM((CW, D), jnp.float32),
            pltpu.VMEM((CW, D), jnp.float32),
            pltpu.SemaphoreType.DMA,
            pltpu.SemaphoreType.DMA,
        ],
    )
    def k(emb_hbm, idx_hbm, out_hbm, idx_v, buf0, buf1, sem0, sem1):
        wid = lax.axis_index("s") * nc + lax.axis_index("c")
        base = wid * (CH * CW)
        pltpu.sync_copy(idx_hbm.at[wid], idx_v)
        bufs = (buf0, buf1)
        sems = (sem0, sem1)
        cps = [None, None]
        cps[0] = pltpu.async_copy(emb_hbm.at[idx_v.at[0]], buf0, sem0)
        for j in range(CH):
            b = j & 1
            cps[b].wait()
            if j + 1 < CH:
                nb = (j + 1) & 1
                cps[nb] = pltpu.async_copy(
                    emb_hbm.at[idx_v.at[j + 1]], bufs[nb], sems[nb])
            pltpu.sync_copy(bufs[b], out_hbm.at[pl.ds(base + j * CW, CW)])

    return k(emb_all, idx3)


TB = 512


def _inproj(e_all, f78, WeT, WfT, b_ih2):
    def body(e_ref, f_ref, we_ref, wf_ref, b_ref, out_ref):
        acc = jnp.dot(e_ref[0].astype(jnp.bfloat16), we_ref[0],
                      preferred_element_type=jnp.float32)
        for i in range(1, NT):
            acc = acc + jnp.dot(e_ref[i].astype(jnp.bfloat16), we_ref[i],
                                preferred_element_type=jnp.float32)
        f = f_ref[...]
        acc = acc + f[:, 0:1] * wf_ref[0:1, :]
        acc = acc + f[:, 1:2] * wf_ref[1:2, :]
        out_ref[...] = acc + b_ref[...]

    return pl.pallas_call(
        body,
        grid=(N // TB,),
        in_specs=[
            pl.BlockSpec((NT, TB, D), lambda j: (0, j, 0)),
            pl.BlockSpec((TB, 2), lambda j: (j, 0)),
            pl.BlockSpec((NT, D, G3), lambda j: (0, 0, 0)),
            pl.BlockSpec((2, G3), lambda j: (0, 0)),
            pl.BlockSpec((1, G3), lambda j: (0, 0)),
        ],
        out_specs=pl.BlockSpec((TB, G3), lambda j: (j, 0)),
        out_shape=jax.ShapeDtypeStruct((N, G3), jnp.float32),
    )(e_all, f78, WeT, WfT, b_ih2)


def _gru(gi3, WhhT, WoutT, b_hh2, b_out2):
    def body(gi_ref, whh_ref, bhh_ref, wout_ref, bo_ref, out_ref, h_ref):
        t = pl.program_id(0)

        @pl.when(t == 0)
        def _():
            h_ref[...] = jnp.zeros_like(h_ref)

        h = h_ref[...]
        gh = jnp.dot(h.astype(jnp.bfloat16), whh_ref[...],
                     preferred_element_type=jnp.float32) + bhh_ref[...]
        g = gi_ref[0]
        r = jax.nn.sigmoid(g[:, :H] + gh[:, :H])
        z = jax.nn.sigmoid(g[:, H:2 * H] + gh[:, H:2 * H])
        n = jnp.tanh(g[:, 2 * H:] + r * gh[:, 2 * H:])
        h_new = (1.0 - z) * n + z * h
        h_ref[...] = h_new
        out_ref[0, :, :] = jnp.dot(
            h_new.astype(jnp.bfloat16), wout_ref[...],
            preferred_element_type=jnp.float32
        ) + bo_ref[...]

    return pl.pallas_call(
        body,
        grid=(L,),
        in_specs=[
            pl.BlockSpec((1, B, G3), lambda t: (t, 0, 0)),
            pl.BlockSpec((H, G3), lambda t: (0, 0)),
            pl.BlockSpec((1, G3), lambda t: (0, 0)),
            pl.BlockSpec((H, V), lambda t: (0, 0)),
            pl.BlockSpec((1, V), lambda t: (0, 0)),
        ],
        out_specs=pl.BlockSpec((1, B, V), lambda t: (t, 0, 0)),
        out_shape=jax.ShapeDtypeStruct((L, B, V), jnp.float32),
        scratch_shapes=[pltpu.VMEM((B, H), jnp.float32)],
    )(gi3, WhhT, b_hh2, WoutT, b_out2)


def kernel(input1, input2, input3, input4, input5, input6, inputs7, inputs8,
           emb1, emb2, emb3, emb4, emb5, emb6,
           W_ih, W_hh, b_ih, b_hh, W_out, b_out):
    idx = jnp.stack([input1, input2, input3, input4, input5, input6])
    idx = idx.astype(jnp.int32).transpose(0, 2, 1).reshape(NT, N)
    idx = idx + jnp.arange(NT, dtype=jnp.int32)[:, None] * V
    idx3 = idx.reshape(NW, CH, CW)

    emb_all = jnp.concatenate([emb1, emb2, emb3, emb4, emb5, emb6], axis=0)
    e_all = _sc_gather(emb_all, idx3).reshape(NT, N, D)

    f78 = jnp.stack(
        [inputs7.T.reshape(N), inputs8.T.reshape(N)], axis=1)

    WeT = W_ih[:, :NT * D].T.reshape(NT, D, G3).astype(jnp.bfloat16)
    WfT = W_ih[:, NT * D:].T
    gi = _inproj(e_all, f78, WeT, WfT, b_ih.reshape(1, G3))

    logits = _gru(gi.reshape(L, B, G3),
                  W_hh.T.astype(jnp.bfloat16), W_out.T.astype(jnp.bfloat16),
                  b_hh.reshape(1, G3), b_out.reshape(1, V))
    return logits.transpose(1, 0, 2)


# speedup vs baseline: 2.2022x; 1.0144x over previous
import functools

import jax
import jax.numpy as jnp
from jax import lax
from jax.experimental import pallas as pl
from jax.experimental.pallas import tpu as pltpu
from jax.experimental.pallas import tpu_sc as plsc

B, L, D, H, V = 256, 50, 128, 1024, 1000
N = B * L
NT = 6
TOT = NT * N
G3 = 3 * H

NW = 32
CH, CW = 20, 120


def _sc_gather(emb_all, idx3):
    mesh = plsc.VectorSubcoreMesh(core_axis_name="c", subcore_axis_name="s")
    info = plsc.get_sparse_core_info()
    nc = info.num_cores

    @functools.partial(
        pl.kernel,
        mesh=mesh,
        out_type=jax.ShapeDtypeStruct((TOT, D), jnp.float32),
        scratch_types=[
            pltpu.VMEM((CH, CW), jnp.int32),
            pltpu.VMEM((CW, D), jnp.float32),
            pltpu.VMEM((CW, D), jnp.float32),
            pltpu.SemaphoreType.DMA,
            pltpu.SemaphoreType.DMA,
        ],
    )
    def k(emb_hbm, idx_hbm, out_hbm, idx_v, buf0, buf1, sem0, sem1):
        wid = lax.axis_index("s") * nc + lax.axis_index("c")
        base = wid * (CH * CW)
        pltpu.sync_copy(idx_hbm.at[wid], idx_v)
        bufs = (buf0, buf1)
        sems = (sem0, sem1)
        cps = [None, None]
        cps[0] = pltpu.async_copy(emb_hbm.at[idx_v.at[0]], buf0, sem0)
        for j in range(CH):
            b = j & 1
            cps[b].wait()
            if j + 1 < CH:
                nb = (j + 1) & 1
                cps[nb] = pltpu.async_copy(
                    emb_hbm.at[idx_v.at[j + 1]], bufs[nb], sems[nb])
            pltpu.sync_copy(bufs[b], out_hbm.at[pl.ds(base + j * CW, CW)])

    return k(emb_all, idx3)


TB = 512


def _inproj(e2d, f128, WeT, Wf128, b_ih2):
    def body(e_ref, f_ref, we_ref, wf_ref, b_ref, out_ref):
        acc = jnp.dot(e_ref[...].astype(jnp.bfloat16), we_ref[...],
                      preferred_element_type=jnp.float32)
        acc = acc + jnp.dot(f_ref[...].astype(jnp.bfloat16), wf_ref[...],
                            preferred_element_type=jnp.float32)
        out_ref[...] = acc + b_ref[...]

    return pl.pallas_call(
        body,
        grid=(N // TB,),
        in_specs=[
            pl.BlockSpec((TB, NT * D), lambda j: (j, 0)),
            pl.BlockSpec((TB, D), lambda j: (j, 0)),
            pl.BlockSpec((NT * D, G3), lambda j: (0, 0)),
            pl.BlockSpec((D, G3), lambda j: (0, 0)),
            pl.BlockSpec((1, G3), lambda j: (0, 0)),
        ],
        out_specs=pl.BlockSpec((TB, G3), lambda j: (j, 0)),
        out_shape=jax.ShapeDtypeStruct((N, G3), jnp.float32),
    )(e2d, f128, WeT, Wf128, b_ih2)


def _gru(gi3, WhhT, WoutT, b_hh2, b_out2):
    def body(gi_ref, whh_ref, bhh_ref, wout_ref, bo_ref, out_ref, h_ref):
        t = pl.program_id(0)

        @pl.when(t == 0)
        def _():
            h_ref[...] = jnp.zeros_like(h_ref)

        h = h_ref[...]
        gh = jnp.dot(h.astype(jnp.bfloat16), whh_ref[...],
                     preferred_element_type=jnp.float32) + bhh_ref[...]
        g = gi_ref[0]
        r = jax.nn.sigmoid(g[:, :H] + gh[:, :H])
        z = jax.nn.sigmoid(g[:, H:2 * H] + gh[:, H:2 * H])
        n = jnp.tanh(g[:, 2 * H:] + r * gh[:, 2 * H:])
        h_new = (1.0 - z) * n + z * h
        h_ref[...] = h_new
        out_ref[0, :, :] = jnp.dot(
            h_new.astype(jnp.bfloat16), wout_ref[...],
            preferred_element_type=jnp.float32
        ) + bo_ref[...]

    return pl.pallas_call(
        body,
        grid=(L,),
        in_specs=[
            pl.BlockSpec((1, B, G3), lambda t: (t, 0, 0)),
            pl.BlockSpec((H, G3), lambda t: (0, 0)),
            pl.BlockSpec((1, G3), lambda t: (0, 0)),
            pl.BlockSpec((H, V), lambda t: (0, 0)),
            pl.BlockSpec((1, V), lambda t: (0, 0)),
        ],
        out_specs=pl.BlockSpec((1, B, V), lambda t: (t, 0, 0)),
        out_shape=jax.ShapeDtypeStruct((L, B, V), jnp.float32),
        scratch_shapes=[pltpu.VMEM((B, H), jnp.float32)],
    )(gi3, WhhT, b_hh2, WoutT, b_out2)


def kernel(input1, input2, input3, input4, input5, input6, inputs7, inputs8,
           emb1, emb2, emb3, emb4, emb5, emb6,
           W_ih, W_hh, b_ih, b_hh, W_out, b_out):
    idx = jnp.stack([input1, input2, input3, input4, input5, input6])
    idx = idx.astype(jnp.int32) + jnp.arange(
        NT, dtype=jnp.int32)[:, None, None] * V
    idx3 = idx.transpose(2, 1, 0).reshape(NW, CH, CW)

    emb_all = jnp.concatenate([emb1, emb2, emb3, emb4, emb5, emb6], axis=0)
    e2d = _sc_gather(emb_all, idx3).reshape(N, NT * D)

    f128 = jnp.pad(
        jnp.stack([inputs7.T.reshape(N), inputs8.T.reshape(N)], axis=1),
        ((0, 0), (0, D - 2)))

    WeT = W_ih[:, :NT * D].T.astype(jnp.bfloat16)
    Wf128 = jnp.pad(W_ih[:, NT * D:].T, ((0, D - 2), (0, 0))
                    ).astype(jnp.bfloat16)
    gi = _inproj(e2d, f128, WeT, Wf128, b_ih.reshape(1, G3))

    logits = _gru(gi.reshape(L, B, G3),
                  W_hh.T.astype(jnp.bfloat16), W_out.T.astype(jnp.bfloat16),
                  b_hh.reshape(1, G3), b_out.reshape(1, V))
    return logits.transpose(1, 0, 2)


# speedup vs baseline: 2.2594x; 1.0260x over previous
import functools

import jax
import jax.numpy as jnp
from jax import lax
from jax.experimental import pallas as pl
from jax.experimental.pallas import tpu as pltpu
from jax.experimental.pallas import tpu_sc as plsc

B, L, D, H, V = 256, 50, 128, 1024, 1000
N = B * L
NT = 6
TOT = NT * N
G3 = 3 * H

NW = 32
CH, CW = 20, 120


def _sc_gather(emb_all, idx3):
    mesh = plsc.VectorSubcoreMesh(core_axis_name="c", subcore_axis_name="s")
    info = plsc.get_sparse_core_info()
    nc = info.num_cores

    @functools.partial(
        pl.kernel,
        mesh=mesh,
        out_type=jax.ShapeDtypeStruct((TOT, D), jnp.float32),
        scratch_types=[
            pltpu.VMEM((CH, CW), jnp.int32),
            pltpu.VMEM((CW, D), jnp.float32),
            pltpu.VMEM((CW, D), jnp.float32),
            pltpu.SemaphoreType.DMA,
            pltpu.SemaphoreType.DMA,
        ],
    )
    def k(emb_hbm, idx_hbm, out_hbm, idx_v, buf0, buf1, sem0, sem1):
        wid = lax.axis_index("s") * nc + lax.axis_index("c")
        base = wid * (CH * CW)
        pltpu.sync_copy(idx_hbm.at[wid], idx_v)
        bufs = (buf0, buf1)
        sems = (sem0, sem1)
        cps = [None, None]
        cps[0] = pltpu.async_copy(emb_hbm.at[idx_v.at[0]], buf0, sem0)
        for j in range(CH):
            b = j & 1
            cps[b].wait()
            if j + 1 < CH:
                nb = (j + 1) & 1
                cps[nb] = pltpu.async_copy(
                    emb_hbm.at[idx_v.at[j + 1]], bufs[nb], sems[nb])
            pltpu.sync_copy(bufs[b], out_hbm.at[pl.ds(base + j * CW, CW)])

    return k(emb_all, idx3)


def _gru_fused(x3, f3, WeT, Wf128, b_ih2, WhhT, b_hh2, WoutT, b_out2):
    def body(x_ref, f_ref, we_ref, wf_ref, bi_ref, whh_ref, bhh_ref,
             wout_ref, bo_ref, out_ref, h_ref):
        t = pl.program_id(0)

        @pl.when(t == 0)
        def _():
            h_ref[...] = jnp.zeros_like(h_ref)

        h = h_ref[...]
        gi = jnp.dot(x_ref[0].astype(jnp.bfloat16), we_ref[...],
                     preferred_element_type=jnp.float32)
        gi = gi + jnp.dot(f_ref[0].astype(jnp.bfloat16), wf_ref[...],
                          preferred_element_type=jnp.float32)
        gi = gi + bi_ref[...]
        gh = jnp.dot(h.astype(jnp.bfloat16), whh_ref[...],
                     preferred_element_type=jnp.float32) + bhh_ref[...]
        r = jax.nn.sigmoid(gi[:, :H] + gh[:, :H])
        z = jax.nn.sigmoid(gi[:, H:2 * H] + gh[:, H:2 * H])
        n = jnp.tanh(gi[:, 2 * H:] + r * gh[:, 2 * H:])
        h_new = (1.0 - z) * n + z * h
        h_ref[...] = h_new
        out_ref[0, :, :] = jnp.dot(
            h_new.astype(jnp.bfloat16), wout_ref[...],
            preferred_element_type=jnp.float32) + bo_ref[...]

    return pl.pallas_call(
        body,
        grid=(L,),
        in_specs=[
            pl.BlockSpec((1, B, NT * D), lambda t: (t, 0, 0)),
            pl.BlockSpec((1, B, D), lambda t: (t, 0, 0)),
            pl.BlockSpec((NT * D, G3), lambda t: (0, 0)),
            pl.BlockSpec((D, G3), lambda t: (0, 0)),
            pl.BlockSpec((1, G3), lambda t: (0, 0)),
            pl.BlockSpec((H, G3), lambda t: (0, 0)),
            pl.BlockSpec((1, G3), lambda t: (0, 0)),
            pl.BlockSpec((H, V), lambda t: (0, 0)),
            pl.BlockSpec((1, V), lambda t: (0, 0)),
        ],
        out_specs=pl.BlockSpec((1, B, V), lambda t: (t, 0, 0)),
        out_shape=jax.ShapeDtypeStruct((L, B, V), jnp.float32),
        scratch_shapes=[pltpu.VMEM((B, H), jnp.float32)],
    )(x3, f3, WeT, Wf128, b_ih2, WhhT, b_hh2, WoutT, b_out2)


def kernel(input1, input2, input3, input4, input5, input6, inputs7, inputs8,
           emb1, emb2, emb3, emb4, emb5, emb6,
           W_ih, W_hh, b_ih, b_hh, W_out, b_out):
    idx = jnp.stack([input1, input2, input3, input4, input5, input6])
    idx = idx.astype(jnp.int32) + jnp.arange(
        NT, dtype=jnp.int32)[:, None, None] * V
    idx3 = idx.transpose(2, 1, 0).reshape(NW, CH, CW)

    emb_all = jnp.concatenate([emb1, emb2, emb3, emb4, emb5, emb6], axis=0)
    x3 = _sc_gather(emb_all, idx3).reshape(L, B, NT * D)

    f3 = jnp.pad(
        jnp.stack([inputs7.T, inputs8.T], axis=2),
        ((0, 0), (0, 0), (0, D - 2)))

    WeT = W_ih[:, :NT * D].T.astype(jnp.bfloat16)
    Wf128 = jnp.pad(W_ih[:, NT * D:].T, ((0, D - 2), (0, 0))
                    ).astype(jnp.bfloat16)

    logits = _gru_fused(x3, f3, WeT, Wf128, b_ih.reshape(1, G3),
                        W_hh.T.astype(jnp.bfloat16), b_hh.reshape(1, G3),
                        W_out.T.astype(jnp.bfloat16), b_out.reshape(1, V))
    return logits.transpose(1, 0, 2)
